# baseline (speedup 1.0000x reference)
import jax
import jax.numpy as jnp
from jax.experimental import pallas as pl
from jax.experimental.pallas import tpu as pltpu


def _combine_body(t_ref, ac_ref, om_ref, x_ref, n_ref, o_ref):
    b = pl.program_id(0)
    tt = t_ref[b]
    c1 = ac_ref[tt]
    c2 = om_ref[tt]
    o_ref[...] = c1 * x_ref[...] + c2 * n_ref[...]


def kernel(x_start, t, noise, sqrt_alphas_cumprod, sqrt_one_minus_alphas_cumprod):
    B = x_start.shape[0]
    rest = x_start.size // B
    LANES = 1024
    ROWS = rest // LANES
    RB = 256

    x3 = x_start.reshape(B, ROWS, LANES)
    n3 = noise.reshape(B, ROWS, LANES)

    smem = pl.BlockSpec(memory_space=pltpu.SMEM)
    blk = pl.BlockSpec((1, RB, LANES), lambda b, r: (b, r, 0))

    out = pl.pallas_call(
        _combine_body,
        grid=(B, ROWS // RB),
        in_specs=[smem, smem, smem, blk, blk],
        out_specs=blk,
        out_shape=jax.ShapeDtypeStruct((B, ROWS, LANES), jnp.float32),
    )(t.astype(jnp.int32), sqrt_alphas_cumprod, sqrt_one_minus_alphas_cumprod,
      x3, n3)
    return out.reshape(x_start.shape)


# speedup vs baseline: 1.0501x; 1.0501x over previous
import jax
import jax.numpy as jnp
from jax.experimental import pallas as pl
from jax.experimental.pallas import tpu as pltpu


def _combine_body(t_ref, ac_ref, om_ref, x_ref, n_ref, o_ref):
    b = pl.program_id(0)
    tt = t_ref[b]
    c1 = ac_ref[tt]
    c2 = om_ref[tt]
    o_ref[...] = c1 * x_ref[...] + c2 * n_ref[...]


def kernel(x_start, t, noise, sqrt_alphas_cumprod, sqrt_one_minus_alphas_cumprod):
    B = x_start.shape[0]
    rest = x_start.size // B
    LANES = 1024
    ROWS = rest // LANES
    RB = 768

    x3 = x_start.reshape(B, ROWS, LANES)
    n3 = noise.reshape(B, ROWS, LANES)

    smem = pl.BlockSpec(memory_space=pltpu.SMEM)
    blk = pl.BlockSpec((1, RB, LANES), lambda b, r: (b, r, 0))

    out = pl.pallas_call(
        _combine_body,
        grid=(B, ROWS // RB),
        in_specs=[smem, smem, smem, blk, blk],
        out_specs=blk,
        out_shape=jax.ShapeDtypeStruct((B, ROWS, LANES), jnp.float32),
    )(t.astype(jnp.int32), sqrt_alphas_cumprod, sqrt_one_minus_alphas_cumprod,
      x3, n3)
    return out.reshape(x_start.shape)


# speedup vs baseline: 4.3560x; 4.1484x over previous
import jax
import jax.numpy as jnp
from jax.experimental import pallas as pl
from jax.experimental.pallas import tpu as pltpu


def _combine_body(t_ref, ac_ref, om_ref, x_ref, n_ref, o_ref):
    b = pl.program_id(0)
    tt = t_ref[b]
    c1 = ac_ref[tt]
    c2 = om_ref[tt]
    o_ref[...] = c1 * x_ref[...] + c2 * n_ref[...]


def kernel(x_start, t, noise, sqrt_alphas_cumprod, sqrt_one_minus_alphas_cumprod):
    B, C, H, W = x_start.shape
    HB = 256

    smem = pl.BlockSpec(memory_space=pltpu.SMEM)
    blk = pl.BlockSpec((1, C, HB, W), lambda b, h: (b, 0, h, 0))

    out = pl.pallas_call(
        _combine_body,
        grid=(B, H // HB),
        in_specs=[smem, smem, smem, blk, blk],
        out_specs=blk,
        out_shape=jax.ShapeDtypeStruct((B, C, H, W), jnp.float32),
    )(t.astype(jnp.int32), sqrt_alphas_cumprod, sqrt_one_minus_alphas_cumprod,
      x_start, noise)
    return out
